# baseline (speedup 1.0000x reference)
import functools

import jax
import jax.numpy as jnp
from jax import lax
from jax.experimental import pallas as pl
from jax.experimental.pallas import tpu as pltpu
from jax.experimental.pallas import tpu_sc as plsc

_H = 12
_K = 8
_HD = 64
_IB = 512


def _mm_body(x_ref, w_ref, b_ref, o_ref):
    acc = jnp.dot(x_ref[...].astype(jnp.bfloat16),
                  w_ref[...].astype(jnp.bfloat16),
                  preferred_element_type=jnp.float32)
    o_ref[...] = acc + b_ref[...]


def _matmul_bias(x, w, b, bm=512):
    m, kdim = x.shape
    n = w.shape[1]
    return pl.pallas_call(
        _mm_body,
        grid=(m // bm,),
        in_specs=[
            pl.BlockSpec((bm, kdim), lambda i: (i, 0)),
            pl.BlockSpec((kdim, n), lambda i: (0, 0)),
            pl.BlockSpec((1, n), lambda i: (0, 0)),
        ],
        out_specs=pl.BlockSpec((bm, n), lambda i: (i, 0)),
        out_shape=jax.ShapeDtypeStruct((m, n), jnp.float32),
    )(x, w, b.reshape(1, n))


def _proj_t_body(bz, x_ref, w_ref, b_ref, keep_ref, o_ref):
    w = w_ref[...].astype(jnp.bfloat16)
    bias = b_ref[...]
    for b in range(bz):
        acc = jnp.dot(x_ref[:, b, :].astype(jnp.bfloat16), w,
                      preferred_element_type=jnp.float32)
        acc = (acc + bias) * keep_ref[:, b:b + 1]
        o_ref[b] = jnp.swapaxes(acc.reshape(_IB, _H, _HD), 0, 1)


def _proj_t(value, w, b, keep2):
    lv, bz, d = value.shape
    nblk = lv // _IB
    last = nblk - 1
    return pl.pallas_call(
        functools.partial(_proj_t_body, bz),
        grid=(nblk + 1,),
        in_specs=[
            pl.BlockSpec((_IB, bz, d), lambda i: (jnp.minimum(i, last), 0, 0)),
            pl.BlockSpec((d, d), lambda i: (0, 0)),
            pl.BlockSpec((1, d), lambda i: (0, 0)),
            pl.BlockSpec((_IB, bz), lambda i: (jnp.minimum(i, last), 0)),
        ],
        out_specs=pl.BlockSpec((bz, _H, _IB, _HD), lambda i: (0, 0, i, 0)),
        out_shape=jax.ShapeDtypeStruct((bz, _H, lv + _IB, _HD), jnp.float32),
    )(value, w, b.reshape(1, d), keep2)


def _pair_body(br2, a_ref, b_ref, o_ref):
    a = a_ref[...]
    ar = jnp.roll(a, -_HD, axis=1)
    br_row = jnp.roll(b_ref[:1], -_HD, axis=1)
    ashift = jnp.concatenate((ar[1:], br_row), axis=0)
    lane = lax.broadcasted_iota(jnp.int32, (br2, 2 * _HD), 1)
    odd = jnp.where(lane < _HD, ar, ashift)
    o_ref[:, 0, :] = a
    o_ref[:, 1, :] = odd


def _pair_build(tbl, br2=1024):
    r = tbl.shape[0]
    in2 = tbl.reshape(r // 2, 2 * _HD)
    r2 = r // 2
    nblk = r2 // br2
    last8 = r2 // 8 - 1
    out3 = pl.pallas_call(
        functools.partial(_pair_body, br2),
        grid=(nblk,),
        in_specs=[
            pl.BlockSpec((br2, 2 * _HD), lambda i: (i, 0)),
            pl.BlockSpec((8, 2 * _HD),
                         lambda i: (jnp.minimum((i + 1) * (br2 // 8), last8), 0)),
        ],
        out_specs=pl.BlockSpec((br2, 2, 2 * _HD), lambda i: (i, 0, 0)),
        out_shape=jax.ShapeDtypeStruct((r2, 2, 2 * _HD), jnp.float32),
    )(in2, in2)
    return out3.reshape(r, 2 * _HD)


def _prep_body(lv, lvp, bz, bm, q_ref, wo_ref, bo_ref, wa_ref, ba_ref, rp_ref,
               sn_ref, g_ref, w0_ref, w1_ref):
    hk = _H * _K
    q = q_ref[...]
    offs = jnp.dot(q, wo_ref[...], preferred_element_type=jnp.float32) + bo_ref[...]
    attn = jnp.dot(q, wa_ref[...], preferred_element_type=jnp.float32) + ba_ref[...]
    e = jnp.exp(attn)
    colg = lax.broadcasted_iota(jnp.int32, (hk, hk), 0) // _K
    colg_t = lax.broadcasted_iota(jnp.int32, (hk, hk), 1) // _K
    bones = (colg == colg_t).astype(jnp.float32)
    seg = jnp.dot(e, bones, preferred_element_type=jnp.float32)
    aw = e / seg
    loc = rp_ref[...] + offs / sn_ref[...]
    grid = loc * 2.0 - 1.0
    x = (grid + 1.0) * ((lv - 1) / 2.0)
    x0f = jnp.floor(x)
    w = x - x0f
    i0 = jnp.clip(x0f, 0.0, lv - 1).astype(jnp.int32)
    m0 = ((x0f >= 0.0) & (x0f <= lv - 1)).astype(jnp.float32)
    m1 = ((x0f + 1.0 >= 0.0) & (x0f + 1.0 <= lv - 1)).astype(jnp.float32)
    c0 = aw * (1.0 - w) * m0
    c1 = aw * w * m1
    sel = (x0f < 0.0).astype(jnp.float32)
    w0_ref[...] = c0 + sel * c1
    w1_ref[...] = (1.0 - sel) * c1
    pid = pl.program_id(0)
    grow = lax.broadcasted_iota(jnp.int32, (bm, hk), 0) + pid * bm
    hcol = lax.broadcasted_iota(jnp.int32, (bm, hk), 1) // _K
    g_ref[...] = ((grow % bz) * _H + hcol) * lvp + i0


def _prep(q2, w_off, b_off, w_attn, b_attn, rp_col, sn_col, lv, lvp, bz,
          bm=1024):
    m, d = q2.shape
    hk = _H * _K
    spec_w = pl.BlockSpec((d, hk), lambda i: (0, 0))
    spec_b = pl.BlockSpec((1, hk), lambda i: (0, 0))
    spec_c = pl.BlockSpec((bm, 1), lambda i: (i, 0))
    spec_o = pl.BlockSpec((bm, hk), lambda i: (i, 0))
    return pl.pallas_call(
        functools.partial(_prep_body, lv, lvp, bz, bm),
        grid=(m // bm,),
        in_specs=[pl.BlockSpec((bm, d), lambda i: (i, 0)), spec_w, spec_b,
                  spec_w, spec_b, spec_c, spec_c],
        out_specs=[spec_o, spec_o, spec_o],
        out_shape=[
            jax.ShapeDtypeStruct((m, hk), jnp.int32),
            jax.ShapeDtypeStruct((m, hk), jnp.float32),
            jax.ShapeDtypeStruct((m, hk), jnp.float32),
        ],
    )(q2, w_off, b_off.reshape(1, hk), w_attn, b_attn.reshape(1, hk),
      rp_col, sn_col)


_CH = 16
_SAM = _CH * _K
_PW = 2 * _HD


def _sc_body(nrows, v_hbm, g_hbm, w0_hbm, w1_hbm, out_hbm,
             idx_v, wt0_v, wt1_v, buf_a, buf_b, buf_c, out_a, out_b, out_c,
             sem_a, sem_b, sem_c, semo_a, semo_b, semo_c):
    nc = plsc.get_sparse_core_info().num_cores
    wid = lax.axis_index("s") * nc + lax.axis_index("c")
    nworkers = 32
    rpw = nrows // nworkers
    nch = rpw // _CH
    ch0 = wid * nch
    row0 = wid * rpw

    pltpu.sync_copy(g_hbm.at[pl.ds(ch0, nch)], idx_v)
    pltpu.sync_copy(w0_hbm.at[pl.ds(ch0, nch)], wt0_v)
    pltpu.sync_copy(w1_hbm.at[pl.ds(ch0, nch)], wt1_v)

    bufs = ((buf_a, sem_a, out_a, semo_a), (buf_b, sem_b, out_b, semo_b),
            (buf_c, sem_c, out_c, semo_c))

    def issue(ci, slot):
        b0, s0 = bufs[slot][:2]
        pltpu.async_copy(v_hbm.at[idx_v.at[ci]], b0, s0)

    def wait_gather(slot):
        b0, s0 = bufs[slot][:2]
        pltpu.make_async_copy(v_hbm.at[pl.ds(0, _SAM)], b0, s0).wait()

    issue(0, 0)
    issue(1, 1)

    def chunk_body(ci, carry):
        nxt = ci + 2

        @pl.when(nxt < nch)
        def _():
            for slot in range(3):
                @pl.when(nxt % 3 == slot)
                def _():
                    issue(nxt, slot)

        for slot in range(3):
            @pl.when(ci % 3 == slot)
            def _():
                b0, _s0, ov, so = bufs[slot]
                wait_gather(slot)

                def pair_body(p, c2):
                    w0v = wt0_v[ci, pl.ds(p * 16, 16)]
                    w1v = wt1_v[ci, pl.ds(p * 16, 16)]
                    for sub in range(2):
                        r = p * 2 + sub
                        s = r * _K
                        for c in range(_HD // 16):
                            acc = jnp.zeros((16,), jnp.float32)
                            for k in range(_K):
                                acc = acc + w0v[sub * _K + k] * b0[s + k, pl.ds(c * 16, 16)]
                                acc = acc + w1v[sub * _K + k] * b0[s + k, pl.ds(_HD + c * 16, 16)]
                            ov[r, pl.ds(c * 16, 16)] = acc
                    return c2

                lax.fori_loop(0, _CH // 2, pair_body, 0)

                @pl.when(ci >= 3)
                def _():
                    pltpu.make_async_copy(out_hbm.at[pl.ds(0, _CH)], ov, so).wait()

                pltpu.async_copy(ov, out_hbm.at[pl.ds(row0 + ci * _CH, _CH)], so)
        return carry

    lax.fori_loop(0, nch, chunk_body, 0)

    for slot in range(3):
        ov, so = bufs[slot][2], bufs[slot][3]
        pltpu.make_async_copy(out_hbm.at[pl.ds(0, _CH)], ov, so).wait()


def _sc_gather(pairs, gf, w0f, w1f):
    nsam = gf.shape[0]
    nrows = nsam // _K
    nch_pw = nrows // 32 // _CH
    gf = gf.reshape(nsam // _SAM, _SAM)
    w0f = w0f.reshape(nsam // _SAM, _SAM)
    w1f = w1f.reshape(nsam // _SAM, _SAM)
    mesh = plsc.VectorSubcoreMesh(core_axis_name="c", subcore_axis_name="s")
    return pl.kernel(
        functools.partial(_sc_body, nrows),
        mesh=mesh,
        compiler_params=pltpu.CompilerParams(use_tc_tiling_on_sc=False),
        out_type=jax.ShapeDtypeStruct((nrows, _HD), jnp.float32),
        scratch_types=[
            pltpu.VMEM((nch_pw, _SAM), jnp.int32),
            pltpu.VMEM((nch_pw, _SAM), jnp.float32),
            pltpu.VMEM((nch_pw, _SAM), jnp.float32),
            pltpu.VMEM((_SAM, _PW), jnp.float32),
            pltpu.VMEM((_SAM, _PW), jnp.float32),
            pltpu.VMEM((_SAM, _PW), jnp.float32),
            pltpu.VMEM((_CH, _HD), jnp.float32),
            pltpu.VMEM((_CH, _HD), jnp.float32),
            pltpu.VMEM((_CH, _HD), jnp.float32),
            pltpu.SemaphoreType.DMA,
            pltpu.SemaphoreType.DMA,
            pltpu.SemaphoreType.DMA,
            pltpu.SemaphoreType.DMA,
            pltpu.SemaphoreType.DMA,
            pltpu.SemaphoreType.DMA,
        ],
    )(pairs, gf, w0f, w1f)


def kernel(query, value, value_key_padding_mask, value_valid_ratio,
           reference_point, snippet_num, W_off, b_off, W_attn, b_attn,
           W_v, b_v, W_o, b_o):
    lq, bz, d = query.shape
    lv = value.shape[0]
    lvp = lv + _IB

    keep2 = 1.0 - value_key_padding_mask.T.astype(jnp.float32)
    vt4 = _proj_t(value, W_v, b_v, keep2)
    tbl = vt4.reshape(bz * _H * lvp, _HD)

    pairs = _pair_build(tbl)

    q2 = query.reshape(lq * bz, d)
    rp_col = reference_point[:, :, 0].T.reshape(lq * bz, 1)
    sn_col = jnp.tile(snippet_num, lq).reshape(lq * bz, 1)
    g, w0, w1 = _prep(q2, W_off, b_off, W_attn, b_attn, rp_col, sn_col,
                      lv, lvp, bz)

    out_rows = _sc_gather(pairs, g.reshape(-1), w0.reshape(-1), w1.reshape(-1))

    out = _matmul_bias(out_rows.reshape(lq * bz, d), W_o, b_o)
    return out.reshape(lq, bz, d)


# speedup vs baseline: 1.0304x; 1.0304x over previous
import functools

import jax
import jax.numpy as jnp
from jax import lax
from jax.experimental import pallas as pl
from jax.experimental.pallas import tpu as pltpu
from jax.experimental.pallas import tpu_sc as plsc

_H = 12
_K = 8
_HD = 64
_IB = 512


def _mm_body(x_ref, w_ref, b_ref, o_ref):
    acc = jnp.dot(x_ref[...].astype(jnp.bfloat16),
                  w_ref[...].astype(jnp.bfloat16),
                  preferred_element_type=jnp.float32)
    o_ref[...] = acc + b_ref[...]


def _matmul_bias(x, w, b, bm=512):
    m, kdim = x.shape
    n = w.shape[1]
    return pl.pallas_call(
        _mm_body,
        grid=(m // bm,),
        in_specs=[
            pl.BlockSpec((bm, kdim), lambda i: (i, 0)),
            pl.BlockSpec((kdim, n), lambda i: (0, 0)),
            pl.BlockSpec((1, n), lambda i: (0, 0)),
        ],
        out_specs=pl.BlockSpec((bm, n), lambda i: (i, 0)),
        out_shape=jax.ShapeDtypeStruct((m, n), jnp.float32),
    )(x, w, b.reshape(1, n))


def _proj_t_body(bz, x_ref, w_ref, b_ref, keep_ref, o_ref):
    w = w_ref[...].astype(jnp.bfloat16)
    bias = b_ref[...]
    for b in range(bz):
        acc = jnp.dot(x_ref[:, b, :].astype(jnp.bfloat16), w,
                      preferred_element_type=jnp.float32)
        acc = (acc + bias) * keep_ref[:, b:b + 1]
        o_ref[b] = jnp.swapaxes(acc.reshape(_IB, _H, _HD), 0, 1)


def _proj_t(value, w, b, keep2):
    lv, bz, d = value.shape
    nblk = lv // _IB
    last = nblk - 1
    return pl.pallas_call(
        functools.partial(_proj_t_body, bz),
        grid=(nblk + 1,),
        in_specs=[
            pl.BlockSpec((_IB, bz, d), lambda i: (jnp.minimum(i, last), 0, 0)),
            pl.BlockSpec((d, d), lambda i: (0, 0)),
            pl.BlockSpec((1, d), lambda i: (0, 0)),
            pl.BlockSpec((_IB, bz), lambda i: (jnp.minimum(i, last), 0)),
        ],
        out_specs=pl.BlockSpec((bz, _H, _IB, _HD), lambda i: (0, 0, i, 0)),
        out_shape=jax.ShapeDtypeStruct((bz, _H, lv + _IB, _HD), jnp.float32),
    )(value, w, b.reshape(1, d), keep2)


def _pair_body(br2, a_ref, b_ref, o_ref):
    a = a_ref[...]
    ar = jnp.roll(a, -_HD, axis=1)
    br_row = jnp.roll(b_ref[:1], -_HD, axis=1)
    ashift = jnp.concatenate((ar[1:], br_row), axis=0)
    lane = lax.broadcasted_iota(jnp.int32, (br2, 2 * _HD), 1)
    odd = jnp.where(lane < _HD, ar, ashift)
    o_ref[:, 0, :] = a
    o_ref[:, 1, :] = odd


def _pair_build(tbl, br2=2048):
    r = tbl.shape[0]
    in2 = tbl.reshape(r // 2, 2 * _HD)
    r2 = r // 2
    nblk = r2 // br2
    last8 = r2 // 8 - 1
    out3 = pl.pallas_call(
        functools.partial(_pair_body, br2),
        grid=(nblk,),
        in_specs=[
            pl.BlockSpec((br2, 2 * _HD), lambda i: (i, 0)),
            pl.BlockSpec((8, 2 * _HD),
                         lambda i: (jnp.minimum((i + 1) * (br2 // 8), last8), 0)),
        ],
        out_specs=pl.BlockSpec((br2, 2, 2 * _HD), lambda i: (i, 0, 0)),
        out_shape=jax.ShapeDtypeStruct((r2, 2, 2 * _HD), jnp.float32),
    )(in2, in2)
    return out3.reshape(r, 2 * _HD)


def _prep_body(lv, lvp, bz, bm, q_ref, wo_ref, bo_ref, wa_ref, ba_ref, rp_ref,
               sn_ref, g_ref, w0_ref, w1_ref):
    hk = _H * _K
    q = q_ref[...]
    offs = jnp.dot(q, wo_ref[...], preferred_element_type=jnp.float32) + bo_ref[...]
    attn = jnp.dot(q, wa_ref[...], preferred_element_type=jnp.float32) + ba_ref[...]
    e = jnp.exp(attn)
    colg = lax.broadcasted_iota(jnp.int32, (hk, hk), 0) // _K
    colg_t = lax.broadcasted_iota(jnp.int32, (hk, hk), 1) // _K
    bones = (colg == colg_t).astype(jnp.float32)
    seg = jnp.dot(e, bones, preferred_element_type=jnp.float32)
    aw = e / seg
    loc = rp_ref[...] + offs / sn_ref[...]
    grid = loc * 2.0 - 1.0
    x = (grid + 1.0) * ((lv - 1) / 2.0)
    x0f = jnp.floor(x)
    w = x - x0f
    i0 = jnp.clip(x0f, 0.0, lv - 1).astype(jnp.int32)
    m0 = ((x0f >= 0.0) & (x0f <= lv - 1)).astype(jnp.float32)
    m1 = ((x0f + 1.0 >= 0.0) & (x0f + 1.0 <= lv - 1)).astype(jnp.float32)
    c0 = aw * (1.0 - w) * m0
    c1 = aw * w * m1
    sel = (x0f < 0.0).astype(jnp.float32)
    w0_ref[...] = c0 + sel * c1
    w1_ref[...] = (1.0 - sel) * c1
    pid = pl.program_id(0)
    grow = lax.broadcasted_iota(jnp.int32, (bm, hk), 0) + pid * bm
    hcol = lax.broadcasted_iota(jnp.int32, (bm, hk), 1) // _K
    g_ref[...] = ((grow % bz) * _H + hcol) * lvp + i0


def _prep(q2, w_off, b_off, w_attn, b_attn, rp_col, sn_col, lv, lvp, bz,
          bm=1024):
    m, d = q2.shape
    hk = _H * _K
    spec_w = pl.BlockSpec((d, hk), lambda i: (0, 0))
    spec_b = pl.BlockSpec((1, hk), lambda i: (0, 0))
    spec_c = pl.BlockSpec((bm, 1), lambda i: (i, 0))
    spec_o = pl.BlockSpec((bm, hk), lambda i: (i, 0))
    return pl.pallas_call(
        functools.partial(_prep_body, lv, lvp, bz, bm),
        grid=(m // bm,),
        in_specs=[pl.BlockSpec((bm, d), lambda i: (i, 0)), spec_w, spec_b,
                  spec_w, spec_b, spec_c, spec_c],
        out_specs=[spec_o, spec_o, spec_o],
        out_shape=[
            jax.ShapeDtypeStruct((m, hk), jnp.int32),
            jax.ShapeDtypeStruct((m, hk), jnp.float32),
            jax.ShapeDtypeStruct((m, hk), jnp.float32),
        ],
    )(q2, w_off, b_off.reshape(1, hk), w_attn, b_attn.reshape(1, hk),
      rp_col, sn_col)


_CH = 16
_SAM = _CH * _K
_PW = 2 * _HD


def _sc_body(nrows, v_hbm, g_hbm, w0_hbm, w1_hbm, out_hbm,
             idx_v, wt0_v, wt1_v, buf_a, buf_b, buf_c, out_a, out_b, out_c,
             sem_a, sem_b, sem_c, semo_a, semo_b, semo_c):
    nc = plsc.get_sparse_core_info().num_cores
    wid = lax.axis_index("s") * nc + lax.axis_index("c")
    nworkers = 32
    rpw = nrows // nworkers
    nch = rpw // _CH
    ch0 = wid * nch
    row0 = wid * rpw

    pltpu.sync_copy(g_hbm.at[pl.ds(ch0, nch)], idx_v)
    pltpu.sync_copy(w0_hbm.at[pl.ds(ch0, nch)], wt0_v)
    pltpu.sync_copy(w1_hbm.at[pl.ds(ch0, nch)], wt1_v)

    bufs = ((buf_a, sem_a, out_a, semo_a), (buf_b, sem_b, out_b, semo_b),
            (buf_c, sem_c, out_c, semo_c))

    def issue(ci, slot):
        b0, s0 = bufs[slot][:2]
        pltpu.async_copy(v_hbm.at[idx_v.at[ci]], b0, s0)

    def wait_gather(slot):
        b0, s0 = bufs[slot][:2]
        pltpu.make_async_copy(v_hbm.at[pl.ds(0, _SAM)], b0, s0).wait()

    issue(0, 0)
    issue(1, 1)

    def chunk_body(ci, carry):
        nxt = ci + 2

        @pl.when(nxt < nch)
        def _():
            for slot in range(3):
                @pl.when(nxt % 3 == slot)
                def _():
                    issue(nxt, slot)

        for slot in range(3):
            @pl.when(ci % 3 == slot)
            def _():
                b0, _s0, ov, so = bufs[slot]
                wait_gather(slot)

                def pair_body(p, c2):
                    w0v = wt0_v[ci, pl.ds(p * 16, 16)]
                    w1v = wt1_v[ci, pl.ds(p * 16, 16)]
                    for sub in range(2):
                        r = p * 2 + sub
                        s = r * _K
                        for c in range(_HD // 16):
                            acc = jnp.zeros((16,), jnp.float32)
                            for k in range(_K):
                                acc = acc + w0v[sub * _K + k] * b0[s + k, pl.ds(c * 16, 16)]
                                acc = acc + w1v[sub * _K + k] * b0[s + k, pl.ds(_HD + c * 16, 16)]
                            ov[r, pl.ds(c * 16, 16)] = acc
                    return c2

                lax.fori_loop(0, _CH // 2, pair_body, 0)

                @pl.when(ci >= 3)
                def _():
                    pltpu.make_async_copy(out_hbm.at[pl.ds(0, _CH)], ov, so).wait()

                pltpu.async_copy(ov, out_hbm.at[pl.ds(row0 + ci * _CH, _CH)], so)
        return carry

    lax.fori_loop(0, nch, chunk_body, 0)

    for slot in range(3):
        ov, so = bufs[slot][2], bufs[slot][3]
        pltpu.make_async_copy(out_hbm.at[pl.ds(0, _CH)], ov, so).wait()


def _sc_gather(pairs, gf, w0f, w1f):
    nsam = gf.shape[0]
    nrows = nsam // _K
    nch_pw = nrows // 32 // _CH
    gf = gf.reshape(nsam // _SAM, _SAM)
    w0f = w0f.reshape(nsam // _SAM, _SAM)
    w1f = w1f.reshape(nsam // _SAM, _SAM)
    mesh = plsc.VectorSubcoreMesh(core_axis_name="c", subcore_axis_name="s")
    return pl.kernel(
        functools.partial(_sc_body, nrows),
        mesh=mesh,
        compiler_params=pltpu.CompilerParams(use_tc_tiling_on_sc=False),
        out_type=jax.ShapeDtypeStruct((nrows, _HD), jnp.float32),
        scratch_types=[
            pltpu.VMEM((nch_pw, _SAM), jnp.int32),
            pltpu.VMEM((nch_pw, _SAM), jnp.float32),
            pltpu.VMEM((nch_pw, _SAM), jnp.float32),
            pltpu.VMEM((_SAM, _PW), jnp.float32),
            pltpu.VMEM((_SAM, _PW), jnp.float32),
            pltpu.VMEM((_SAM, _PW), jnp.float32),
            pltpu.VMEM((_CH, _HD), jnp.float32),
            pltpu.VMEM((_CH, _HD), jnp.float32),
            pltpu.VMEM((_CH, _HD), jnp.float32),
            pltpu.SemaphoreType.DMA,
            pltpu.SemaphoreType.DMA,
            pltpu.SemaphoreType.DMA,
            pltpu.SemaphoreType.DMA,
            pltpu.SemaphoreType.DMA,
            pltpu.SemaphoreType.DMA,
        ],
    )(pairs, gf, w0f, w1f)


def kernel(query, value, value_key_padding_mask, value_valid_ratio,
           reference_point, snippet_num, W_off, b_off, W_attn, b_attn,
           W_v, b_v, W_o, b_o):
    lq, bz, d = query.shape
    lv = value.shape[0]
    lvp = lv + _IB

    keep2 = 1.0 - value_key_padding_mask.T.astype(jnp.float32)
    vt4 = _proj_t(value, W_v, b_v, keep2)
    tbl = vt4.reshape(bz * _H * lvp, _HD)

    pairs = _pair_build(tbl)

    q2 = query.reshape(lq * bz, d)
    rp_col = reference_point[:, :, 0].T.reshape(lq * bz, 1)
    sn_col = jnp.tile(snippet_num, lq).reshape(lq * bz, 1)
    g, w0, w1 = _prep(q2, W_off, b_off, W_attn, b_attn, rp_col, sn_col,
                      lv, lvp, bz)

    out_rows = _sc_gather(pairs, g.reshape(-1), w0.reshape(-1), w1.reshape(-1))

    out = _matmul_bias(out_rows.reshape(lq * bz, d), W_o, b_o)
    return out.reshape(lq, bz, d)


# speedup vs baseline: 1.0516x; 1.0206x over previous
import functools

import jax
import jax.numpy as jnp
from jax import lax
from jax.experimental import pallas as pl
from jax.experimental.pallas import tpu as pltpu
from jax.experimental.pallas import tpu_sc as plsc

_H = 12
_K = 8
_HD = 64
_IB = 512


def _mm_body(x_ref, w_ref, b_ref, o_ref):
    acc = jnp.dot(x_ref[...].astype(jnp.bfloat16),
                  w_ref[...].astype(jnp.bfloat16),
                  preferred_element_type=jnp.float32)
    o_ref[...] = acc + b_ref[...]


def _matmul_bias(x, w, b, bm=512):
    m, kdim = x.shape
    n = w.shape[1]
    return pl.pallas_call(
        _mm_body,
        grid=(m // bm,),
        in_specs=[
            pl.BlockSpec((bm, kdim), lambda i: (i, 0)),
            pl.BlockSpec((kdim, n), lambda i: (0, 0)),
            pl.BlockSpec((1, n), lambda i: (0, 0)),
        ],
        out_specs=pl.BlockSpec((bm, n), lambda i: (i, 0)),
        out_shape=jax.ShapeDtypeStruct((m, n), jnp.float32),
    )(x, w, b.reshape(1, n))


def _proj_t_body(bz, x_ref, w_ref, b_ref, keep_ref, o_ref):
    w = w_ref[...].astype(jnp.bfloat16)
    bias = b_ref[...]
    for b in range(bz):
        acc = jnp.dot(x_ref[:, b, :].astype(jnp.bfloat16), w,
                      preferred_element_type=jnp.float32)
        acc = (acc + bias) * keep_ref[:, b:b + 1]
        o_ref[b] = jnp.swapaxes(acc.reshape(_IB, _H, _HD), 0, 1)


def _proj_t(value, w, b, keep2):
    lv, bz, d = value.shape
    nblk = lv // _IB
    last = nblk - 1
    return pl.pallas_call(
        functools.partial(_proj_t_body, bz),
        grid=(nblk + 1,),
        in_specs=[
            pl.BlockSpec((_IB, bz, d), lambda i: (jnp.minimum(i, last), 0, 0)),
            pl.BlockSpec((d, d), lambda i: (0, 0)),
            pl.BlockSpec((1, d), lambda i: (0, 0)),
            pl.BlockSpec((_IB, bz), lambda i: (jnp.minimum(i, last), 0)),
        ],
        out_specs=pl.BlockSpec((bz, _H, _IB, _HD), lambda i: (0, 0, i, 0)),
        out_shape=jax.ShapeDtypeStruct((bz, _H, lv + _IB, _HD), jnp.float32),
    )(value, w, b.reshape(1, d), keep2)


def _pair_body(br2, a_ref, b_ref, o_ref):
    a = a_ref[...]
    ar = jnp.roll(a, -_HD, axis=1)
    br_row = jnp.roll(b_ref[:1], -_HD, axis=1)
    ashift = jnp.concatenate((ar[1:], br_row), axis=0)
    lane = lax.broadcasted_iota(jnp.int32, (br2, 2 * _HD), 1)
    odd = jnp.where(lane < _HD, ar, ashift)
    o_ref[:, 0, :] = a
    o_ref[:, 1, :] = odd


def _pair_build(tbl, br2=4096):
    r = tbl.shape[0]
    in2 = tbl.reshape(r // 2, 2 * _HD)
    r2 = r // 2
    nblk = r2 // br2
    last8 = r2 // 8 - 1
    out3 = pl.pallas_call(
        functools.partial(_pair_body, br2),
        grid=(nblk,),
        in_specs=[
            pl.BlockSpec((br2, 2 * _HD), lambda i: (i, 0)),
            pl.BlockSpec((8, 2 * _HD),
                         lambda i: (jnp.minimum((i + 1) * (br2 // 8), last8), 0)),
        ],
        out_specs=pl.BlockSpec((br2, 2, 2 * _HD), lambda i: (i, 0, 0)),
        out_shape=jax.ShapeDtypeStruct((r2, 2, 2 * _HD), jnp.float32),
    )(in2, in2)
    return out3.reshape(r, 2 * _HD)


def _prep_body(lv, lvp, bz, bm, q_ref, wo_ref, bo_ref, wa_ref, ba_ref, rp_ref,
               sn_ref, g_ref, w0_ref, w1_ref):
    hk = _H * _K
    q = q_ref[...]
    offs = jnp.dot(q, wo_ref[...], preferred_element_type=jnp.float32) + bo_ref[...]
    attn = jnp.dot(q, wa_ref[...], preferred_element_type=jnp.float32) + ba_ref[...]
    e = jnp.exp(attn)
    colg = lax.broadcasted_iota(jnp.int32, (hk, hk), 0) // _K
    colg_t = lax.broadcasted_iota(jnp.int32, (hk, hk), 1) // _K
    bones = (colg == colg_t).astype(jnp.float32)
    seg = jnp.dot(e, bones, preferred_element_type=jnp.float32)
    aw = e / seg
    loc = rp_ref[...] + offs / sn_ref[...]
    grid = loc * 2.0 - 1.0
    x = (grid + 1.0) * ((lv - 1) / 2.0)
    x0f = jnp.floor(x)
    w = x - x0f
    i0 = jnp.clip(x0f, 0.0, lv - 1).astype(jnp.int32)
    m0 = ((x0f >= 0.0) & (x0f <= lv - 1)).astype(jnp.float32)
    m1 = ((x0f + 1.0 >= 0.0) & (x0f + 1.0 <= lv - 1)).astype(jnp.float32)
    c0 = aw * (1.0 - w) * m0
    c1 = aw * w * m1
    sel = (x0f < 0.0).astype(jnp.float32)
    w0_ref[...] = c0 + sel * c1
    w1_ref[...] = (1.0 - sel) * c1
    pid = pl.program_id(0)
    grow = lax.broadcasted_iota(jnp.int32, (bm, hk), 0) + pid * bm
    hcol = lax.broadcasted_iota(jnp.int32, (bm, hk), 1) // _K
    g_ref[...] = ((grow % bz) * _H + hcol) * lvp + i0


def _prep(q2, w_off, b_off, w_attn, b_attn, rp_col, sn_col, lv, lvp, bz,
          bm=1024):
    m, d = q2.shape
    hk = _H * _K
    spec_w = pl.BlockSpec((d, hk), lambda i: (0, 0))
    spec_b = pl.BlockSpec((1, hk), lambda i: (0, 0))
    spec_c = pl.BlockSpec((bm, 1), lambda i: (i, 0))
    spec_o = pl.BlockSpec((bm, hk), lambda i: (i, 0))
    return pl.pallas_call(
        functools.partial(_prep_body, lv, lvp, bz, bm),
        grid=(m // bm,),
        in_specs=[pl.BlockSpec((bm, d), lambda i: (i, 0)), spec_w, spec_b,
                  spec_w, spec_b, spec_c, spec_c],
        out_specs=[spec_o, spec_o, spec_o],
        out_shape=[
            jax.ShapeDtypeStruct((m, hk), jnp.int32),
            jax.ShapeDtypeStruct((m, hk), jnp.float32),
            jax.ShapeDtypeStruct((m, hk), jnp.float32),
        ],
    )(q2, w_off, b_off.reshape(1, hk), w_attn, b_attn.reshape(1, hk),
      rp_col, sn_col)


_CH = 16
_SAM = _CH * _K
_PW = 2 * _HD


def _sc_body(nrows, v_hbm, g_hbm, w0_hbm, w1_hbm, out_hbm,
             idx_v, wt0_v, wt1_v, buf_a, buf_b, buf_c, out_a, out_b, out_c,
             sem_a, sem_b, sem_c, semo_a, semo_b, semo_c):
    nc = plsc.get_sparse_core_info().num_cores
    wid = lax.axis_index("s") * nc + lax.axis_index("c")
    nworkers = 32
    rpw = nrows // nworkers
    nch = rpw // _CH
    ch0 = wid * nch
    row0 = wid * rpw

    pltpu.sync_copy(g_hbm.at[pl.ds(ch0, nch)], idx_v)
    pltpu.sync_copy(w0_hbm.at[pl.ds(ch0, nch)], wt0_v)
    pltpu.sync_copy(w1_hbm.at[pl.ds(ch0, nch)], wt1_v)

    bufs = ((buf_a, sem_a, out_a, semo_a), (buf_b, sem_b, out_b, semo_b),
            (buf_c, sem_c, out_c, semo_c))

    def issue(ci, slot):
        b0, s0 = bufs[slot][:2]
        pltpu.async_copy(v_hbm.at[idx_v.at[ci]], b0, s0)

    def wait_gather(slot):
        b0, s0 = bufs[slot][:2]
        pltpu.make_async_copy(v_hbm.at[pl.ds(0, _SAM)], b0, s0).wait()

    issue(0, 0)
    issue(1, 1)

    def chunk_body(ci, carry):
        nxt = ci + 2

        @pl.when(nxt < nch)
        def _():
            for slot in range(3):
                @pl.when(nxt % 3 == slot)
                def _():
                    issue(nxt, slot)

        for slot in range(3):
            @pl.when(ci % 3 == slot)
            def _():
                b0, _s0, ov, so = bufs[slot]
                wait_gather(slot)

                def pair_body(p, c2):
                    w0v = wt0_v[ci, pl.ds(p * 16, 16)]
                    w1v = wt1_v[ci, pl.ds(p * 16, 16)]
                    for sub in range(2):
                        r = p * 2 + sub
                        s = r * _K
                        for c in range(_HD // 16):
                            acc = jnp.zeros((16,), jnp.float32)
                            for k in range(_K):
                                acc = acc + w0v[sub * _K + k] * b0[s + k, pl.ds(c * 16, 16)]
                                acc = acc + w1v[sub * _K + k] * b0[s + k, pl.ds(_HD + c * 16, 16)]
                            ov[r, pl.ds(c * 16, 16)] = acc
                    return c2

                lax.fori_loop(0, _CH // 2, pair_body, 0)

                @pl.when(ci >= 3)
                def _():
                    pltpu.make_async_copy(out_hbm.at[pl.ds(0, _CH)], ov, so).wait()

                pltpu.async_copy(ov, out_hbm.at[pl.ds(row0 + ci * _CH, _CH)], so)
        return carry

    lax.fori_loop(0, nch, chunk_body, 0)

    for slot in range(3):
        ov, so = bufs[slot][2], bufs[slot][3]
        pltpu.make_async_copy(out_hbm.at[pl.ds(0, _CH)], ov, so).wait()


def _sc_gather(pairs, gf, w0f, w1f):
    nsam = gf.shape[0]
    nrows = nsam // _K
    nch_pw = nrows // 32 // _CH
    gf = gf.reshape(nsam // _SAM, _SAM)
    w0f = w0f.reshape(nsam // _SAM, _SAM)
    w1f = w1f.reshape(nsam // _SAM, _SAM)
    mesh = plsc.VectorSubcoreMesh(core_axis_name="c", subcore_axis_name="s")
    return pl.kernel(
        functools.partial(_sc_body, nrows),
        mesh=mesh,
        compiler_params=pltpu.CompilerParams(use_tc_tiling_on_sc=False),
        out_type=jax.ShapeDtypeStruct((nrows, _HD), jnp.float32),
        scratch_types=[
            pltpu.VMEM((nch_pw, _SAM), jnp.int32),
            pltpu.VMEM((nch_pw, _SAM), jnp.float32),
            pltpu.VMEM((nch_pw, _SAM), jnp.float32),
            pltpu.VMEM((_SAM, _PW), jnp.float32),
            pltpu.VMEM((_SAM, _PW), jnp.float32),
            pltpu.VMEM((_SAM, _PW), jnp.float32),
            pltpu.VMEM((_CH, _HD), jnp.float32),
            pltpu.VMEM((_CH, _HD), jnp.float32),
            pltpu.VMEM((_CH, _HD), jnp.float32),
            pltpu.SemaphoreType.DMA,
            pltpu.SemaphoreType.DMA,
            pltpu.SemaphoreType.DMA,
            pltpu.SemaphoreType.DMA,
            pltpu.SemaphoreType.DMA,
            pltpu.SemaphoreType.DMA,
        ],
    )(pairs, gf, w0f, w1f)


def kernel(query, value, value_key_padding_mask, value_valid_ratio,
           reference_point, snippet_num, W_off, b_off, W_attn, b_attn,
           W_v, b_v, W_o, b_o):
    lq, bz, d = query.shape
    lv = value.shape[0]
    lvp = lv + _IB

    keep2 = 1.0 - value_key_padding_mask.T.astype(jnp.float32)
    vt4 = _proj_t(value, W_v, b_v, keep2)
    tbl = vt4.reshape(bz * _H * lvp, _HD)

    pairs = _pair_build(tbl)

    q2 = query.reshape(lq * bz, d)
    rp_col = reference_point[:, :, 0].T.reshape(lq * bz, 1)
    sn_col = jnp.tile(snippet_num, lq).reshape(lq * bz, 1)
    g, w0, w1 = _prep(q2, W_off, b_off, W_attn, b_attn, rp_col, sn_col,
                      lv, lvp, bz)

    out_rows = _sc_gather(pairs, g.reshape(-1), w0.reshape(-1), w1.reshape(-1))

    out = _matmul_bias(out_rows.reshape(lq * bz, d), W_o, b_o)
    return out.reshape(lq, bz, d)
